# baseline (speedup 1.0000x reference)
import functools

import jax
import jax.numpy as jnp
from jax import lax
from jax.experimental import pallas as pl
from jax.experimental.pallas import tpu as pltpu
from jax.experimental.pallas import tpu_sc as plsc

N = 10000
E = 320000
DIM = 128
H = 16
F = 8

NC = 2
NS = 16
CHUNK = 40
PER_SUB = E // (NC * NS)
NCHUNK = PER_SUB // CHUNK
NPAIR = NCHUNK // 2 - 1
ZROWS = 40

_HIGH = lax.Precision.HIGHEST


def _dot(a, b):
    return lax.dot_general(a, b, (((1,), (0,)), ((), ())), precision=_HIGH,
                           preferred_element_type=jnp.float32)


def _tc_pre(x, gat_Wp, att_src_b, att_dst_b, S):
    BN = 1000

    def body(x_ref, w_ref, as_ref, ad_ref, s_ref, tg_ref, ts_ref, adst_ref):
        xb = x_ref[...]
        h_t = _dot(xb, w_ref[...])
        tg_ref[:, :DIM] = h_t
        tg_ref[:, DIM:] = _dot(h_t * as_ref[...], s_ref[...])
        adst_ref[...] = _dot(h_t * ad_ref[...], s_ref[...])
        ts_ref[:, :DIM] = xb
        ts_ref[:, DIM:] = jnp.ones((BN, H), jnp.float32)

    return pl.pallas_call(
        body,
        grid=(N // BN,),
        in_specs=[
            pl.BlockSpec((BN, DIM), lambda i: (i, 0)),
            pl.BlockSpec((DIM, DIM), lambda i: (0, 0)),
            pl.BlockSpec((1, DIM), lambda i: (0, 0)),
            pl.BlockSpec((1, DIM), lambda i: (0, 0)),
            pl.BlockSpec((DIM, H), lambda i: (0, 0)),
        ],
        out_specs=[
            pl.BlockSpec((BN, AW), lambda i: (i, 0)),
            pl.BlockSpec((BN, AW), lambda i: (i, 0)),
            pl.BlockSpec((BN, H), lambda i: (i, 0)),
        ],
        out_shape=[
            jax.ShapeDtypeStruct((N, AW), jnp.float32),
            jax.ShapeDtypeStruct((N, AW), jnp.float32),
            jax.ShapeDtypeStruct((N, H), jnp.float32),
        ],
    )(x, gat_Wp, att_src_b, att_dst_b, S)


AW = DIM + H


def _sc_gat(tG, a_dst, src3, dst3):
    mesh = plsc.VectorSubcoreMesh(core_axis_name="c", subcore_axis_name="s")

    @functools.partial(
        pl.kernel,
        mesh=mesh,
        out_type=jax.ShapeDtypeStruct((NC * N, AW), jnp.float32),
        compiler_params=pltpu.CompilerParams(use_tc_tiling_on_sc=False),
        scratch_types=[
            pltpu.VMEM((ZROWS, AW), jnp.float32),
            pltpu.VMEM((CHUNK, AW), jnp.float32),
            pltpu.VMEM((CHUNK, AW), jnp.float32),
            pltpu.VMEM((CHUNK, H), jnp.float32),
            pltpu.VMEM((CHUNK, H), jnp.float32),
            pltpu.VMEM((NCHUNK, CHUNK), jnp.int32),
            pltpu.VMEM((NCHUNK, CHUNK), jnp.int32),
            pltpu.VMEM_SHARED((N, AW), jnp.float32),
            pltpu.SemaphoreType.DMA,
            pltpu.SemaphoreType.DMA,
            pltpu.SemaphoreType.DMA,
            pltpu.SemaphoreType.DMA,
        ],
    )
    def k(tg_hbm, ad_hbm, si_hbm, di_hbm, out_hbm,
          zw, cbuf0, cbuf1, dbuf0, dbuf1,
          sidx, didx, acc, gsem0, gsem1, ssem0, ssem1):
        cid = lax.axis_index("c")
        sid = lax.axis_index("s")
        wid = cid * NS + sid

        @pl.loop(0, ZROWS)
        def _(r):
            @pl.loop(0, AW, step=16)
            def _(cc):
                zw[pl.ds(r, 1), pl.ds(cc, 16)] = jnp.zeros((1, 16), jnp.float32)

        pltpu.sync_copy(si_hbm.at[wid], sidx)
        pltpu.sync_copy(di_hbm.at[wid], didx)

        @pl.loop(sid * ZROWS, N, step=NS * ZROWS)
        def _(row):
            pltpu.sync_copy(zw, acc.at[pl.ds(row, ZROWS)])

        plsc.subcore_barrier()

        def wait_fused(sem):
            pltpu.make_async_copy(tg_hbm.at[pl.ds(0, CHUNK)], cbuf0, sem).wait()

        def wait_narrow(sem):
            pltpu.make_async_copy(ad_hbm.at[pl.ds(0, CHUNK)], dbuf0, sem).wait()

        def load(j, cb, db, sem):
            pltpu.async_copy(tg_hbm.at[sidx.at[j]], cb, sem)
            pltpu.async_copy(ad_hbm.at[didx.at[j]], db, sem)

        def wait_load(sem):
            wait_fused(sem)
            wait_narrow(sem)

        def compute(cb, db):
            @plsc.parallel_loop(0, CHUNK, unroll=4)
            def _(i):
                t = cb[pl.ds(i, 1), pl.ds(DIM, H)] + db[pl.ds(i, 1), :]
                ex = jnp.exp(jnp.maximum(t, t * 0.2))
                cb[pl.ds(i, 1), pl.ds(DIM, H)] = ex
                for f in range(F):
                    sl = (pl.ds(i, 1), pl.ds(16 * f, 16))
                    cb[sl] = cb[sl] * ex

        def scatter(j, cb, sem):
            pltpu.async_copy(cb, acc.at[didx.at[j]], sem, add=True)

        load(0, cbuf0, dbuf0, gsem0)

        @pl.loop(0, NPAIR)
        def _(kk):
            j = 2 * kk
            wait_load(gsem0)

            @pl.when(kk > 0)
            def _():
                wait_fused(ssem1)

            load(j + 1, cbuf1, dbuf1, gsem1)
            compute(cbuf0, dbuf0)
            scatter(j, cbuf0, ssem0)

            wait_load(gsem1)
            wait_fused(ssem0)
            load(j + 2, cbuf0, dbuf0, gsem0)
            compute(cbuf1, dbuf1)
            scatter(j + 1, cbuf1, ssem1)

        wait_load(gsem0)
        wait_fused(ssem1)
        load(NCHUNK - 1, cbuf1, dbuf1, gsem1)
        compute(cbuf0, dbuf0)
        scatter(NCHUNK - 2, cbuf0, ssem0)
        wait_load(gsem1)
        wait_fused(ssem0)
        compute(cbuf1, dbuf1)
        scatter(NCHUNK - 1, cbuf1, ssem1)
        wait_fused(ssem1)

        plsc.subcore_barrier()

        @pl.loop(sid * ZROWS, N, step=NS * ZROWS)
        def _(row):
            pltpu.sync_copy(acc.at[pl.ds(row, ZROWS)],
                            out_hbm.at[pl.ds(cid * N + row, ZROWS)])

    return k(tG, a_dst, src3, dst3)


def _sc_sage(tS, src3, dst3):
    mesh = plsc.VectorSubcoreMesh(core_axis_name="c", subcore_axis_name="s")

    @functools.partial(
        pl.kernel,
        mesh=mesh,
        out_type=jax.ShapeDtypeStruct((NC * N, AW), jnp.float32),
        compiler_params=pltpu.CompilerParams(use_tc_tiling_on_sc=False),
        scratch_types=[
            pltpu.VMEM((ZROWS, AW), jnp.float32),
            pltpu.VMEM((CHUNK, AW), jnp.float32),
            pltpu.VMEM((CHUNK, AW), jnp.float32),
            pltpu.VMEM((NCHUNK, CHUNK), jnp.int32),
            pltpu.VMEM((NCHUNK, CHUNK), jnp.int32),
            pltpu.VMEM_SHARED((N, AW), jnp.float32),
            pltpu.SemaphoreType.DMA,
            pltpu.SemaphoreType.DMA,
            pltpu.SemaphoreType.DMA,
            pltpu.SemaphoreType.DMA,
        ],
    )
    def k(ts_hbm, si_hbm, di_hbm, out_hbm,
          zw, cbuf0, cbuf1, sidx, didx, acc, gsem0, gsem1, ssem0, ssem1):
        cid = lax.axis_index("c")
        sid = lax.axis_index("s")
        wid = cid * NS + sid

        @pl.loop(0, ZROWS)
        def _(r):
            @pl.loop(0, AW, step=16)
            def _(cc):
                zw[pl.ds(r, 1), pl.ds(cc, 16)] = jnp.zeros((1, 16), jnp.float32)

        pltpu.sync_copy(si_hbm.at[wid], sidx)
        pltpu.sync_copy(di_hbm.at[wid], didx)

        @pl.loop(sid * ZROWS, N, step=NS * ZROWS)
        def _(row):
            pltpu.sync_copy(zw, acc.at[pl.ds(row, ZROWS)])

        plsc.subcore_barrier()

        def wait_fused(sem):
            pltpu.make_async_copy(ts_hbm.at[pl.ds(0, CHUNK)], cbuf0, sem).wait()

        pltpu.async_copy(ts_hbm.at[sidx.at[0]], cbuf0, gsem0)

        @pl.loop(0, NPAIR)
        def _(kk):
            j = 2 * kk
            wait_fused(gsem0)

            @pl.when(kk > 0)
            def _():
                wait_fused(ssem1)

            pltpu.async_copy(ts_hbm.at[sidx.at[j + 1]], cbuf1, gsem1)
            pltpu.async_copy(cbuf0, acc.at[didx.at[j]], ssem0, add=True)

            wait_fused(gsem1)
            wait_fused(ssem0)
            pltpu.async_copy(ts_hbm.at[sidx.at[j + 2]], cbuf0, gsem0)
            pltpu.async_copy(cbuf1, acc.at[didx.at[j + 1]], ssem1, add=True)

        wait_fused(gsem0)
        wait_fused(ssem1)
        pltpu.async_copy(ts_hbm.at[sidx.at[NCHUNK - 1]], cbuf1, gsem1)
        pltpu.async_copy(cbuf0, acc.at[didx.at[NCHUNK - 2]], ssem0, add=True)
        wait_fused(gsem1)
        wait_fused(ssem0)
        pltpu.async_copy(cbuf1, acc.at[didx.at[NCHUNK - 1]], ssem1, add=True)
        wait_fused(ssem1)

        plsc.subcore_barrier()

        @pl.loop(sid * ZROWS, N, step=NS * ZROWS)
        def _(row):
            pltpu.sync_copy(acc.at[pl.ds(row, ZROWS)],
                            out_hbm.at[pl.ds(cid * N + row, ZROWS)])

    return k(tS, src3, dst3)


def _tc_post(gf, sf, tG, a_dst, x, R,
             gat_bias_t, sage_Wl, sage_Wr, sage_bias, PWg, PWs, proj_b,
             ln_g, ln_b):
    BN = 1000
    NB = N // BN

    def body(g0_ref, g1_ref, s0_ref, s1_ref, tg_ref, ad_ref, x_ref,
             r_ref, gb_ref, wl_ref, wr_ref, sb_ref, pwg_ref, pws_ref, pb_ref,
             lg_ref, lb_ref, o_ref):
        t = tg_ref[:, DIM:] + ad_ref[...]
        ex_self = jnp.exp(jnp.maximum(t, t * 0.2))
        numer_tot = (g0_ref[:, :DIM] + g1_ref[:, :DIM]
                     + tg_ref[:, :DIM] * _dot(ex_self, r_ref[...]))
        den_tot = _dot(g0_ref[:, DIM:] + g1_ref[:, DIM:] + ex_self, r_ref[...])
        gat_t = numer_tot / den_tot + gb_ref[...]
        cntb = _dot(s0_ref[:, DIM:] + s1_ref[:, DIM:], r_ref[...])
        mean = (s0_ref[:, :DIM] + s1_ref[:, :DIM]) / jnp.maximum(cntb, 1.0)
        sage_out = _dot(mean, wl_ref[...]) + _dot(x_ref[...], wr_ref[...]) + sb_ref[...]
        y = _dot(gat_t, pwg_ref[...]) + _dot(sage_out, pws_ref[...]) + pb_ref[...] + x_ref[...]
        mu = jnp.mean(y, axis=1, keepdims=True)
        d = y - mu
        var = jnp.mean(d * d, axis=1, keepdims=True)
        o_ref[...] = d * jax.lax.rsqrt(var + 1e-5) * lg_ref[...] + lb_ref[...]

    row_spec = lambda w: pl.BlockSpec((BN, w), lambda i: (i, 0))
    off_spec = lambda w: pl.BlockSpec((BN, w), lambda i: (i + NB, 0))
    full_spec = lambda a, b: pl.BlockSpec((a, b), lambda i: (0, 0))

    return pl.pallas_call(
        body,
        grid=(NB,),
        in_specs=[
            row_spec(AW), off_spec(AW), row_spec(AW), off_spec(AW),
            row_spec(AW), row_spec(H), row_spec(DIM),
            full_spec(H, DIM),
            full_spec(1, DIM), full_spec(DIM, DIM), full_spec(DIM, DIM),
            full_spec(1, DIM), full_spec(DIM, DIM), full_spec(DIM, DIM),
            full_spec(1, DIM), full_spec(1, DIM), full_spec(1, DIM),
        ],
        out_specs=pl.BlockSpec((BN, DIM), lambda i: (i, 0)),
        out_shape=jax.ShapeDtypeStruct((N, DIM), jnp.float32),
    )(gf, gf, sf, sf, tG, a_dst, x, R,
      gat_bias_t, sage_Wl, sage_Wr, sage_bias, PWg, PWs, proj_b, ln_g, ln_b)


@jax.jit
def kernel(x, edge_index, gat_W, att_src, att_dst, gat_bias,
           sage_Wl, sage_Wr, sage_bias, proj_W, proj_b, ln_g, ln_b):
    idx_p = jnp.array([(p % H) * F + p // H for p in range(DIM)], jnp.int32)
    gat_Wp = gat_W[:, idx_p]
    att_src_b = att_src.T.reshape(1, DIM)
    att_dst_b = att_dst.T.reshape(1, DIM)
    S = jnp.tile(jnp.eye(H, dtype=jnp.float32), (F, 1))
    R = S.T
    gat_bias_t = gat_bias[idx_p].reshape(1, DIM)
    PWg = proj_W[:DIM][idx_p]
    PWs = proj_W[DIM:]

    tG, tS, a_dst = _tc_pre(x, gat_Wp, att_src_b, att_dst_b, S)

    src3 = edge_index[0].reshape(NC * NS, NCHUNK, CHUNK)
    dst3 = edge_index[1].reshape(NC * NS, NCHUNK, CHUNK)
    gf = _sc_gat(tG, a_dst, src3, dst3)
    sf = _sc_sage(tS, src3, dst3)

    return _tc_post(gf, sf, tG, a_dst, x, R,
                    gat_bias_t, sage_Wl, sage_Wr, sage_bias.reshape(1, DIM),
                    PWg, PWs, proj_b.reshape(1, DIM), ln_g.reshape(1, DIM),
                    ln_b.reshape(1, DIM))


# speedup vs baseline: 1.0010x; 1.0010x over previous
import functools

import jax
import jax.numpy as jnp
from jax import lax
from jax.experimental import pallas as pl
from jax.experimental.pallas import tpu as pltpu
from jax.experimental.pallas import tpu_sc as plsc

N = 10000
E = 320000
DIM = 128
H = 16
F = 8

NC = 2
NS = 16
CHUNK = 40
PER_SUB = E // (NC * NS)
NCHUNK = PER_SUB // CHUNK
NPAIR = NCHUNK // 2 - 1
ZROWS = 40

_HIGH = lax.Precision.HIGHEST


def _dot(a, b):
    return lax.dot_general(a, b, (((1,), (0,)), ((), ())), precision=_HIGH,
                           preferred_element_type=jnp.float32)


def _tc_pre(x, gat_Wp, att_src_b, att_dst_b, S):
    BN = 1000

    def body(x_ref, w_ref, as_ref, ad_ref, s_ref, tg_ref, ts_ref, adst_ref):
        xb = x_ref[...]
        h_t = _dot(xb, w_ref[...])
        tg_ref[:, :DIM] = h_t
        tg_ref[:, DIM:] = _dot(h_t * as_ref[...], s_ref[...])
        adst_ref[...] = _dot(h_t * ad_ref[...], s_ref[...])
        ts_ref[:, :DIM] = xb
        ts_ref[:, DIM:] = jnp.ones((BN, H), jnp.float32)

    return pl.pallas_call(
        body,
        grid=(N // BN,),
        in_specs=[
            pl.BlockSpec((BN, DIM), lambda i: (i, 0)),
            pl.BlockSpec((DIM, DIM), lambda i: (0, 0)),
            pl.BlockSpec((1, DIM), lambda i: (0, 0)),
            pl.BlockSpec((1, DIM), lambda i: (0, 0)),
            pl.BlockSpec((DIM, H), lambda i: (0, 0)),
        ],
        out_specs=[
            pl.BlockSpec((BN, AW), lambda i: (i, 0)),
            pl.BlockSpec((BN, AW), lambda i: (i, 0)),
            pl.BlockSpec((BN, H), lambda i: (i, 0)),
        ],
        out_shape=[
            jax.ShapeDtypeStruct((N, AW), jnp.float32),
            jax.ShapeDtypeStruct((N, AW), jnp.float32),
            jax.ShapeDtypeStruct((N, H), jnp.float32),
        ],
    )(x, gat_Wp, att_src_b, att_dst_b, S)


AW = DIM + H


def _sc_gat(tG, a_dst, src3, dst3):
    mesh = plsc.VectorSubcoreMesh(core_axis_name="c", subcore_axis_name="s")

    @functools.partial(
        pl.kernel,
        mesh=mesh,
        out_type=jax.ShapeDtypeStruct((NC * N, AW), jnp.float32),
        compiler_params=pltpu.CompilerParams(use_tc_tiling_on_sc=False),
        scratch_types=[
            pltpu.VMEM((ZROWS, AW), jnp.float32),
            pltpu.VMEM((CHUNK, AW), jnp.float32),
            pltpu.VMEM((CHUNK, AW), jnp.float32),
            pltpu.VMEM((CHUNK, H), jnp.float32),
            pltpu.VMEM((CHUNK, H), jnp.float32),
            pltpu.VMEM((NCHUNK, CHUNK), jnp.int32),
            pltpu.VMEM((NCHUNK, CHUNK), jnp.int32),
            pltpu.VMEM_SHARED((N, AW), jnp.float32),
            pltpu.SemaphoreType.DMA,
            pltpu.SemaphoreType.DMA,
            pltpu.SemaphoreType.DMA,
            pltpu.SemaphoreType.DMA,
        ],
    )
    def k(tg_hbm, ad_hbm, si_hbm, di_hbm, out_hbm,
          zw, cbuf0, cbuf1, dbuf0, dbuf1,
          sidx, didx, acc, gsem0, gsem1, ssem0, ssem1):
        cid = lax.axis_index("c")
        sid = lax.axis_index("s")
        wid = cid * NS + sid

        @pl.loop(0, ZROWS)
        def _(r):
            @pl.loop(0, AW, step=16)
            def _(cc):
                zw[pl.ds(r, 1), pl.ds(cc, 16)] = jnp.zeros((1, 16), jnp.float32)

        pltpu.sync_copy(si_hbm.at[wid], sidx)
        pltpu.sync_copy(di_hbm.at[wid], didx)

        @pl.loop(sid * ZROWS, N, step=NS * ZROWS)
        def _(row):
            pltpu.sync_copy(zw, acc.at[pl.ds(row, ZROWS)])

        plsc.subcore_barrier()

        def wait_fused(sem):
            pltpu.make_async_copy(tg_hbm.at[pl.ds(0, CHUNK)], cbuf0, sem).wait()

        def wait_narrow(sem):
            pltpu.make_async_copy(ad_hbm.at[pl.ds(0, CHUNK)], dbuf0, sem).wait()

        def load(j, cb, db, sem):
            pltpu.async_copy(tg_hbm.at[sidx.at[j]], cb, sem)
            pltpu.async_copy(ad_hbm.at[didx.at[j]], db, sem)

        def wait_load(sem):
            wait_fused(sem)
            wait_narrow(sem)

        def compute(cb, db):
            @plsc.parallel_loop(0, CHUNK, unroll=2)
            def _(i):
                t = cb[pl.ds(i, 1), pl.ds(DIM, H)] + db[pl.ds(i, 1), :]
                ex = jnp.exp(jnp.maximum(t, t * 0.2))
                cb[pl.ds(i, 1), pl.ds(DIM, H)] = ex
                for f in range(F):
                    sl = (pl.ds(i, 1), pl.ds(16 * f, 16))
                    cb[sl] = cb[sl] * ex

        def scatter(j, cb, sem):
            pltpu.async_copy(cb, acc.at[didx.at[j]], sem, add=True)

        load(0, cbuf0, dbuf0, gsem0)

        @pl.loop(0, NPAIR)
        def _(kk):
            j = 2 * kk
            wait_load(gsem0)

            @pl.when(kk > 0)
            def _():
                wait_fused(ssem1)

            load(j + 1, cbuf1, dbuf1, gsem1)
            compute(cbuf0, dbuf0)
            scatter(j, cbuf0, ssem0)

            wait_load(gsem1)
            wait_fused(ssem0)
            load(j + 2, cbuf0, dbuf0, gsem0)
            compute(cbuf1, dbuf1)
            scatter(j + 1, cbuf1, ssem1)

        wait_load(gsem0)
        wait_fused(ssem1)
        load(NCHUNK - 1, cbuf1, dbuf1, gsem1)
        compute(cbuf0, dbuf0)
        scatter(NCHUNK - 2, cbuf0, ssem0)
        wait_load(gsem1)
        wait_fused(ssem0)
        compute(cbuf1, dbuf1)
        scatter(NCHUNK - 1, cbuf1, ssem1)
        wait_fused(ssem1)

        plsc.subcore_barrier()

        @pl.loop(sid * ZROWS, N, step=NS * ZROWS)
        def _(row):
            pltpu.sync_copy(acc.at[pl.ds(row, ZROWS)],
                            out_hbm.at[pl.ds(cid * N + row, ZROWS)])

    return k(tG, a_dst, src3, dst3)


def _sc_sage(tS, src3, dst3):
    mesh = plsc.VectorSubcoreMesh(core_axis_name="c", subcore_axis_name="s")

    @functools.partial(
        pl.kernel,
        mesh=mesh,
        out_type=jax.ShapeDtypeStruct((NC * N, AW), jnp.float32),
        compiler_params=pltpu.CompilerParams(use_tc_tiling_on_sc=False),
        scratch_types=[
            pltpu.VMEM((ZROWS, AW), jnp.float32),
            pltpu.VMEM((CHUNK, AW), jnp.float32),
            pltpu.VMEM((CHUNK, AW), jnp.float32),
            pltpu.VMEM((NCHUNK, CHUNK), jnp.int32),
            pltpu.VMEM((NCHUNK, CHUNK), jnp.int32),
            pltpu.VMEM_SHARED((N, AW), jnp.float32),
            pltpu.SemaphoreType.DMA,
            pltpu.SemaphoreType.DMA,
            pltpu.SemaphoreType.DMA,
            pltpu.SemaphoreType.DMA,
        ],
    )
    def k(ts_hbm, si_hbm, di_hbm, out_hbm,
          zw, cbuf0, cbuf1, sidx, didx, acc, gsem0, gsem1, ssem0, ssem1):
        cid = lax.axis_index("c")
        sid = lax.axis_index("s")
        wid = cid * NS + sid

        @pl.loop(0, ZROWS)
        def _(r):
            @pl.loop(0, AW, step=16)
            def _(cc):
                zw[pl.ds(r, 1), pl.ds(cc, 16)] = jnp.zeros((1, 16), jnp.float32)

        pltpu.sync_copy(si_hbm.at[wid], sidx)
        pltpu.sync_copy(di_hbm.at[wid], didx)

        @pl.loop(sid * ZROWS, N, step=NS * ZROWS)
        def _(row):
            pltpu.sync_copy(zw, acc.at[pl.ds(row, ZROWS)])

        plsc.subcore_barrier()

        def wait_fused(sem):
            pltpu.make_async_copy(ts_hbm.at[pl.ds(0, CHUNK)], cbuf0, sem).wait()

        pltpu.async_copy(ts_hbm.at[sidx.at[0]], cbuf0, gsem0)

        @pl.loop(0, NPAIR)
        def _(kk):
            j = 2 * kk
            wait_fused(gsem0)

            @pl.when(kk > 0)
            def _():
                wait_fused(ssem1)

            pltpu.async_copy(ts_hbm.at[sidx.at[j + 1]], cbuf1, gsem1)
            pltpu.async_copy(cbuf0, acc.at[didx.at[j]], ssem0, add=True)

            wait_fused(gsem1)
            wait_fused(ssem0)
            pltpu.async_copy(ts_hbm.at[sidx.at[j + 2]], cbuf0, gsem0)
            pltpu.async_copy(cbuf1, acc.at[didx.at[j + 1]], ssem1, add=True)

        wait_fused(gsem0)
        wait_fused(ssem1)
        pltpu.async_copy(ts_hbm.at[sidx.at[NCHUNK - 1]], cbuf1, gsem1)
        pltpu.async_copy(cbuf0, acc.at[didx.at[NCHUNK - 2]], ssem0, add=True)
        wait_fused(gsem1)
        wait_fused(ssem0)
        pltpu.async_copy(cbuf1, acc.at[didx.at[NCHUNK - 1]], ssem1, add=True)
        wait_fused(ssem1)

        plsc.subcore_barrier()

        @pl.loop(sid * ZROWS, N, step=NS * ZROWS)
        def _(row):
            pltpu.sync_copy(acc.at[pl.ds(row, ZROWS)],
                            out_hbm.at[pl.ds(cid * N + row, ZROWS)])

    return k(tS, src3, dst3)


def _tc_post(gf, sf, tG, a_dst, x, R,
             gat_bias_t, sage_Wl, sage_Wr, sage_bias, PWg, PWs, proj_b,
             ln_g, ln_b):
    BN = 1000
    NB = N // BN

    def body(g0_ref, g1_ref, s0_ref, s1_ref, tg_ref, ad_ref, x_ref,
             r_ref, gb_ref, wl_ref, wr_ref, sb_ref, pwg_ref, pws_ref, pb_ref,
             lg_ref, lb_ref, o_ref):
        t = tg_ref[:, DIM:] + ad_ref[...]
        ex_self = jnp.exp(jnp.maximum(t, t * 0.2))
        numer_tot = (g0_ref[:, :DIM] + g1_ref[:, :DIM]
                     + tg_ref[:, :DIM] * _dot(ex_self, r_ref[...]))
        den_tot = _dot(g0_ref[:, DIM:] + g1_ref[:, DIM:] + ex_self, r_ref[...])
        gat_t = numer_tot / den_tot + gb_ref[...]
        cntb = _dot(s0_ref[:, DIM:] + s1_ref[:, DIM:], r_ref[...])
        mean = (s0_ref[:, :DIM] + s1_ref[:, :DIM]) / jnp.maximum(cntb, 1.0)
        sage_out = _dot(mean, wl_ref[...]) + _dot(x_ref[...], wr_ref[...]) + sb_ref[...]
        y = _dot(gat_t, pwg_ref[...]) + _dot(sage_out, pws_ref[...]) + pb_ref[...] + x_ref[...]
        mu = jnp.mean(y, axis=1, keepdims=True)
        d = y - mu
        var = jnp.mean(d * d, axis=1, keepdims=True)
        o_ref[...] = d * jax.lax.rsqrt(var + 1e-5) * lg_ref[...] + lb_ref[...]

    row_spec = lambda w: pl.BlockSpec((BN, w), lambda i: (i, 0))
    off_spec = lambda w: pl.BlockSpec((BN, w), lambda i: (i + NB, 0))
    full_spec = lambda a, b: pl.BlockSpec((a, b), lambda i: (0, 0))

    return pl.pallas_call(
        body,
        grid=(NB,),
        in_specs=[
            row_spec(AW), off_spec(AW), row_spec(AW), off_spec(AW),
            row_spec(AW), row_spec(H), row_spec(DIM),
            full_spec(H, DIM),
            full_spec(1, DIM), full_spec(DIM, DIM), full_spec(DIM, DIM),
            full_spec(1, DIM), full_spec(DIM, DIM), full_spec(DIM, DIM),
            full_spec(1, DIM), full_spec(1, DIM), full_spec(1, DIM),
        ],
        out_specs=pl.BlockSpec((BN, DIM), lambda i: (i, 0)),
        out_shape=jax.ShapeDtypeStruct((N, DIM), jnp.float32),
    )(gf, gf, sf, sf, tG, a_dst, x, R,
      gat_bias_t, sage_Wl, sage_Wr, sage_bias, PWg, PWs, proj_b, ln_g, ln_b)


@jax.jit
def kernel(x, edge_index, gat_W, att_src, att_dst, gat_bias,
           sage_Wl, sage_Wr, sage_bias, proj_W, proj_b, ln_g, ln_b):
    idx_p = jnp.array([(p % H) * F + p // H for p in range(DIM)], jnp.int32)
    gat_Wp = gat_W[:, idx_p]
    att_src_b = att_src.T.reshape(1, DIM)
    att_dst_b = att_dst.T.reshape(1, DIM)
    S = jnp.tile(jnp.eye(H, dtype=jnp.float32), (F, 1))
    R = S.T
    gat_bias_t = gat_bias[idx_p].reshape(1, DIM)
    PWg = proj_W[:DIM][idx_p]
    PWs = proj_W[DIM:]

    tG, tS, a_dst = _tc_pre(x, gat_Wp, att_src_b, att_dst_b, S)

    src3 = edge_index[0].reshape(NC * NS, NCHUNK, CHUNK)
    dst3 = edge_index[1].reshape(NC * NS, NCHUNK, CHUNK)
    gf = _sc_gat(tG, a_dst, src3, dst3)
    sf = _sc_sage(tS, src3, dst3)

    return _tc_post(gf, sf, tG, a_dst, x, R,
                    gat_bias_t, sage_Wl, sage_Wr, sage_bias.reshape(1, DIM),
                    PWg, PWs, proj_b.reshape(1, DIM), ln_g.reshape(1, DIM),
                    ln_b.reshape(1, DIM))
